# Initial kernel scaffold; baseline (speedup 1.0000x reference)
import jax
import jax.numpy as jnp
from jax.experimental import pallas as pl


def kernel(tokens, token_emb_w, pos_enc, att_anchors_0, att_thresh_0, att_table_0, ffn_anchors_0, ffn_thresh_0, ffn_table_0, att_anchors_1, att_thresh_1, att_table_1, ffn_anchors_1, ffn_thresh_1, ffn_table_1, unemb_anchors, unemb_thresh, unemb_table):
    raise NotImplementedError("write your pallas kernel here")


# speedup vs baseline: 1.2839x; 1.2839x over previous
import jax
import jax.numpy as jnp
from jax import lax
from jax.experimental import pallas as pl
from jax.experimental.pallas import tpu as pltpu
from jax.experimental.pallas import tpu_sc as plsc

_VOCAB = 1000
_DIM = 256
_CTX = 2048
_POS = 16
_DET = 64
_ANCH = 8
_DET_ATT = 128
_SEQ = _CTX * _DIM
_POW2 = 256
_NW = 32
_TPW = _CTX // _NW
_ZR128 = _SEQ // 128
_VPAD = 1024


def _wid():
    return lax.axis_index("c") * 16 + lax.axis_index("s")


def _accum_rows(acc_v, buf_v, row0, width):
    @pl.loop(0, _TPW, unroll=2)
    def _(t):
        for c in range(width // 16):
            sl = pl.ds(c * 16, 16)
            plsc.addupdate(acc_v.at[t, sl], buf_v[row0 + t, sl])


def _lut_accum(tab_hbm, ridx_v, acc_v, b0_v, b1_v, sem0, sem1, ndet, width):
    bufs = (b0_v, b1_v)
    sems = (sem0, sem1)

    def start(pair, b):
        pltpu.async_copy(
            tab_hbm.at[ridx_v.at[pl.ds((1 + 2 * pair) * _TPW, 2 * _TPW)]],
            bufs[b], sems[b])

    def wait(b):
        pltpu.make_async_copy(tab_hbm.at[ridx_v.at[pl.ds(0, 2 * _TPW)]],
                              bufs[b], sems[b]).wait()

    npair = (ndet - 2) // 2
    nq = npair // 2

    pltpu.async_copy(tab_hbm.at[ridx_v.at[pl.ds(0, _TPW)]],
                     acc_v, sem0).wait()
    start(0, 0)
    start(1, 1)

    @pl.loop(0, nq)
    def _(q):
        p0 = 2 * q
        wait(0)
        _accum_rows(acc_v, b0_v, 0, width)
        _accum_rows(acc_v, b0_v, _TPW, width)

        @pl.when(p0 + 2 < npair)
        def _():
            start(p0 + 2, 0)

        wait(1)
        _accum_rows(acc_v, b1_v, 0, width)
        _accum_rows(acc_v, b1_v, _TPW, width)

        @pl.when(p0 + 3 < npair)
        def _():
            start(p0 + 3, 1)

    if npair % 2:
        wait(0)
        _accum_rows(acc_v, b0_v, 0, width)
        _accum_rows(acc_v, b0_v, _TPW, width)
    pltpu.async_copy(tab_hbm.at[ridx_v.at[pl.ds((ndet - 1) * _TPW, _TPW)]],
                     b1_v.at[pl.ds(0, _TPW)], sem1).wait()
    _accum_rows(acc_v, b1_v, 0, width)


def _residual_add(z_v, acc_v):
    @pl.loop(0, _TPW, unroll=2)
    def _(t):
        for c in range(_DIM // 16):
            sl = pl.ds(c * 16, 16)
            acc_v[t, sl] = z_v[t, sl] + acc_v[t, sl]


def _embed_body(tok_hbm, emb_hbm, z_hbm, tok_v, rows_v, sem):
    base = _wid() * _TPW
    pltpu.sync_copy(tok_hbm.at[pl.ds(base, _TPW)], tok_v)
    pltpu.async_copy(emb_hbm.at[tok_v], rows_v, sem).wait()
    pltpu.sync_copy(rows_v, z_hbm.at[pl.ds(base, _TPW)])


def _att_codes_body(z128_hbm, pos_hbm, aaT_hbm, atT_hbm, r_hbm,
                    aaT_v, atT_v, rowidx_v, vals_v, pos_v, out_v, sem):
    lanes = lax.iota(jnp.int32, 16)
    wid = _wid()
    base = wid * _TPW
    pltpu.sync_copy(aaT_hbm, aaT_v)
    pltpu.sync_copy(atT_hbm, atT_v)
    pltpu.sync_copy(pos_hbm.at[pl.ds(base, _TPW)], pos_v)

    accs = [jnp.zeros((16,), jnp.int32) for _ in range(_DET_ATT // 16)]
    for j in range(_ANCH):
        for g in range(_DET_ATT // 16):
            a = aaT_v[j, pl.ds(g * 16, 16)]
            row = jnp.minimum(lax.shift_right_logical(a, 7), _ZR128 - 1)
            rowidx_v[pl.ds(g * 16, 16)] = row
        pltpu.async_copy(z128_hbm.at[rowidx_v], vals_v, sem).wait()
        for g in range(_DET_ATT // 16):
            a = aaT_v[j, pl.ds(g * 16, 16)]
            th = atT_v[j, pl.ds(g * 16, 16)]
            val = plsc.load_gather(
                vals_v, [g * 16 + lanes, lax.bitwise_and(a, 127)])
            bit = jnp.logical_and(val > th, a < _SEQ)
            accs[g] = accs[g] + jnp.where(bit, 1 << j, 0)

    for g in range(_DET_ATT // 16):
        dvec = g * 16 + lanes
        rowbase = accs[g] + dvec * _POW2

        @pl.loop(0, _TPW)
        def _init(t, rowbase=rowbase, dvec=dvec):
            plsc.store_scatter(out_v, [dvec * _TPW + t], rowbase)

        for j in range(_ANCH):
            a = aaT_v[j, pl.ds(g * 16, 16)]
            th = atT_v[j, pl.ds(g * 16, 16)]
            ispos = a >= _SEQ
            col = jnp.clip(a - _SEQ, 0, _POS - 1)
            w = jnp.where(ispos, 1 << j, 0)

            @pl.when(jnp.max(ispos.astype(jnp.int32)) > 0)
            def _pos(col=col, th=th, w=w, dvec=dvec):
                @pl.loop(0, _TPW)
                def _(t):
                    tsp = jnp.full((16,), t, jnp.int32)
                    val = plsc.load_gather(pos_v, [tsp, col])
                    plsc.addupdate_scatter(out_v, [dvec * _TPW + t],
                                           jnp.where(val > th, w, 0))

    pltpu.sync_copy(out_v, r_hbm.at[wid])


def _row_codes_into(z_v, aT_v, tT_v, out_v, ndet):
    lanes = lax.iota(jnp.int32, 16)
    for tg in range(_TPW // 16):
        tvec = tg * 16 + lanes

        @pl.loop(0, ndet)
        def _(d, tvec=tvec):
            dsp = jnp.full((16,), d, jnp.int32)
            code = jnp.zeros((16,), jnp.int32)
            for j in range(_ANCH):
                jsp = jnp.full((16,), j, jnp.int32)
                a16 = plsc.load_gather(aT_v, [jsp, dsp])
                th16 = plsc.load_gather(tT_v, [jsp, dsp])
                val = plsc.load_gather(z_v, [tvec, a16])
                code = code + jnp.where(val > th16, 1 << j, 0)
            plsc.store_scatter(out_v, [dsp * _TPW + tvec],
                               code + d * _POW2)


def _att_sum_body(z_hbm, r_hbm, tab_hbm, faT_hbm, ftT_hbm,
                  zout_hbm, rffn_hbm,
                  z_v, acc_v, b0_v, b1_v, ridx_v, faT_v, ftT_v, rout_v,
                  sem0, sem1):
    wid = _wid()
    base = wid * _TPW
    pltpu.sync_copy(z_hbm.at[pl.ds(base, _TPW)], z_v)
    pltpu.sync_copy(r_hbm.at[wid], ridx_v)
    pltpu.sync_copy(faT_hbm, faT_v)
    pltpu.sync_copy(ftT_hbm, ftT_v)
    _lut_accum(tab_hbm, ridx_v, acc_v, b0_v, b1_v, sem0, sem1,
               _DET_ATT, _DIM)
    _residual_add(z_v, acc_v)
    pltpu.sync_copy(acc_v, zout_hbm.at[pl.ds(base, _TPW)])
    _row_codes_into(acc_v, faT_v, ftT_v, rout_v, _DET)
    pltpu.sync_copy(rout_v, rffn_hbm.at[wid])


def _ffn_sum_body(z_hbm, r_hbm, tab_hbm, zout_hbm,
                  z_v, acc_v, b0_v, b1_v, ridx_v, sem0, sem1):
    wid = _wid()
    base = wid * _TPW
    pltpu.sync_copy(z_hbm.at[pl.ds(base, _TPW)], z_v)
    pltpu.sync_copy(r_hbm.at[wid], ridx_v)
    _lut_accum(tab_hbm, ridx_v, acc_v, b0_v, b1_v, sem0, sem1, _DET, _DIM)
    _residual_add(z_v, acc_v)
    pltpu.sync_copy(acc_v, zout_hbm.at[pl.ds(base, _TPW)])


def _ffn_sum_codes_body(z_hbm, r_hbm, tab_hbm, uaT_hbm, utT_hbm,
                        zout_hbm, ru_hbm,
                        z_v, acc_v, b0_v, b1_v, ridx_v, uaT_v, utT_v, rout_v,
                        sem0, sem1):
    wid = _wid()
    base = wid * _TPW
    pltpu.sync_copy(z_hbm.at[pl.ds(base, _TPW)], z_v)
    pltpu.sync_copy(r_hbm.at[wid], ridx_v)
    pltpu.sync_copy(uaT_hbm, uaT_v)
    pltpu.sync_copy(utT_hbm, utT_v)
    _lut_accum(tab_hbm, ridx_v, acc_v, b0_v, b1_v, sem0, sem1, _DET, _DIM)
    _residual_add(z_v, acc_v)
    pltpu.sync_copy(acc_v, zout_hbm.at[pl.ds(base, _TPW)])
    _row_codes_into(acc_v, uaT_v, utT_v, rout_v, _DET)
    pltpu.sync_copy(rout_v, ru_hbm.at[wid])


def _unemb_body(r_hbm, tab_hbm, out_hbm, acc_v, b0_v, b1_v, ridx_v,
                sem0, sem1):
    wid = _wid()
    base = wid * _TPW
    CH = 16
    NCH = _TPW // CH
    pltpu.sync_copy(r_hbm.at[wid], ridx_v)
    bufs = (b0_v, b1_v)
    sems = (sem0, sem1)

    def start_dyn(d, q, b):
        pltpu.async_copy(tab_hbm.at[ridx_v.at[pl.ds(d * _TPW + q * CH, CH)]],
                         bufs[b], sems[b])

    def wait(b):
        pltpu.make_async_copy(tab_hbm.at[ridx_v.at[pl.ds(0, CH)]],
                              bufs[b], sems[b]).wait()

    for q in range(NCH):
        pltpu.async_copy(tab_hbm.at[ridx_v.at[pl.ds(q * CH, CH)]],
                         acc_v.at[pl.ds(q * CH, CH)], sems[0])
    pltpu.make_async_copy(tab_hbm.at[ridx_v.at[pl.ds(0, _TPW)]],
                          acc_v, sems[0]).wait()

    start_dyn(1, 0, 0)
    start_dyn(1, 1, 1)

    @pl.loop(1, _DET)
    def _(d):
        for q in range(NCH):
            b = q % 2
            wait(b)

            @pl.loop(0, CH, unroll=2)
            def _(t, q=q, b=b):
                for c in range(_VPAD // 16):
                    sl = pl.ds(c * 16, 16)
                    plsc.addupdate(acc_v.at[q * CH + t, sl], bufs[b][t, sl])

            nd = jnp.where(q + 2 < NCH, d, d + 1)
            nq = (q + 2) % NCH

            @pl.when(nd < _DET)
            def _(nd=nd, nq=nq, b=b):
                pltpu.async_copy(
                    tab_hbm.at[ridx_v.at[pl.ds(nd * _TPW + nq * CH, CH)]],
                    bufs[b], sems[b])

    pltpu.sync_copy(acc_v, out_hbm.at[pl.ds(base, _TPW)])


def kernel(tokens, token_emb_w, pos_enc,
           att_anchors_0, att_thresh_0, att_table_0,
           ffn_anchors_0, ffn_thresh_0, ffn_table_0,
           att_anchors_1, att_thresh_1, att_table_1,
           ffn_anchors_1, ffn_thresh_1, ffn_table_1,
           unemb_anchors, unemb_thresh, unemb_table):
    f32 = jnp.float32
    i32 = jnp.int32
    S = jax.ShapeDtypeStruct
    VM = pltpu.VMEM
    SEM = pltpu.SemaphoreType.DMA
    mesh = plsc.VectorSubcoreMesh(core_axis_name="c", subcore_axis_name="s")
    cp = pltpu.CompilerParams(needs_layout_passes=False)

    embed = pl.kernel(
        _embed_body, out_type=S((_CTX, _DIM), f32), mesh=mesh,
        compiler_params=cp,
        scratch_types=[VM((_TPW,), i32), VM((_TPW, _DIM), f32), SEM])

    att_codes = pl.kernel(
        _att_codes_body, out_type=S((_NW, _DET_ATT * _TPW), i32), mesh=mesh,
        compiler_params=cp,
        scratch_types=[VM((_ANCH, _DET_ATT), i32), VM((_ANCH, _DET_ATT), f32),
                       VM((_DET_ATT,), i32), VM((_DET_ATT, 128), f32),
                       VM((_TPW, _POS), f32), VM((_DET_ATT * _TPW,), i32), SEM])

    att_sum = pl.kernel(
        _att_sum_body,
        out_type=(S((_CTX, _DIM), f32), S((_NW, _DET * _TPW), i32)), mesh=mesh,
        compiler_params=cp,
        scratch_types=[VM((_TPW, _DIM), f32), VM((_TPW, _DIM), f32),
                       VM((2 * _TPW, _DIM), f32), VM((2 * _TPW, _DIM), f32),
                       VM((_DET_ATT * _TPW,), i32),
                       VM((_ANCH, _DET), i32), VM((_ANCH, _DET), f32),
                       VM((_DET * _TPW,), i32), SEM, SEM])

    ffn_sum = pl.kernel(
        _ffn_sum_body, out_type=S((_CTX, _DIM), f32), mesh=mesh,
        compiler_params=cp,
        scratch_types=[VM((_TPW, _DIM), f32), VM((_TPW, _DIM), f32),
                       VM((2 * _TPW, _DIM), f32), VM((2 * _TPW, _DIM), f32),
                       VM((_DET * _TPW,), i32), SEM, SEM])

    ffn_sum_codes = pl.kernel(
        _ffn_sum_codes_body,
        out_type=(S((_CTX, _DIM), f32), S((_NW, _DET * _TPW), i32)), mesh=mesh,
        compiler_params=cp,
        scratch_types=[VM((_TPW, _DIM), f32), VM((_TPW, _DIM), f32),
                       VM((2 * _TPW, _DIM), f32), VM((2 * _TPW, _DIM), f32),
                       VM((_DET * _TPW,), i32),
                       VM((_ANCH, _DET), i32), VM((_ANCH, _DET), f32),
                       VM((_DET * _TPW,), i32), SEM, SEM])

    unemb = pl.kernel(
        _unemb_body, out_type=S((_CTX, _VPAD), f32), mesh=mesh,
        compiler_params=cp,
        scratch_types=[VM((_TPW, _VPAD), f32),
                       VM((16, _VPAD), f32), VM((16, _VPAD), f32),
                       VM((_DET * _TPW,), i32), SEM, SEM])

    tok = tokens.reshape(_CTX)
    aaT = (att_anchors_0.T, att_anchors_1.T)
    atT = (att_thresh_0.T, att_thresh_1.T)
    faT = (ffn_anchors_0.T, ffn_anchors_1.T)
    ftT = (ffn_thresh_0.T, ffn_thresh_1.T)
    atab = (att_table_0.reshape(-1, _DIM), att_table_1.reshape(-1, _DIM))
    ftab = (ffn_table_0.reshape(-1, _DIM), ffn_table_1.reshape(-1, _DIM))
    utab = jnp.pad(unemb_table.reshape(-1, _VOCAB),
                   ((0, 0), (0, _VPAD - _VOCAB)))

    z = embed(tok, token_emb_w)
    ra0 = att_codes(z.reshape(_ZR128, 128), pos_enc, aaT[0], atT[0])
    z, rf0 = att_sum(z, ra0, atab[0], faT[0], ftT[0])
    z = ffn_sum(z, rf0, ftab[0])
    ra1 = att_codes(z.reshape(_ZR128, 128), pos_enc, aaT[1], atT[1])
    z, rf1 = att_sum(z, ra1, atab[1], faT[1], ftT[1])
    z, ru = ffn_sum_codes(z, rf1, ftab[1], unemb_anchors.T, unemb_thresh.T)
    logits = unemb(ru, utab)
    return logits[:, :_VOCAB].reshape(1, _CTX, _VOCAB)


# speedup vs baseline: 1.7089x; 1.3310x over previous
import jax
import jax.numpy as jnp
from jax import lax
from jax.experimental import pallas as pl
from jax.experimental.pallas import tpu as pltpu
from jax.experimental.pallas import tpu_sc as plsc

_VOCAB = 1000
_DIM = 256
_CTX = 2048
_POS = 16
_DET = 64
_ANCH = 8
_DET_ATT = 128
_SEQ = _CTX * _DIM
_POW2 = 256
_NW = 32
_TPW = _CTX // _NW
_ZR128 = _SEQ // 128
_VPAD = 1024


def _wid():
    return lax.axis_index("c") * 16 + lax.axis_index("s")


def _accum_rows(acc_v, buf_v, row0, width):
    @pl.loop(0, _TPW, unroll=2)
    def _(t):
        for c in range(width // 16):
            sl = pl.ds(c * 16, 16)
            plsc.addupdate(acc_v.at[t, sl], buf_v[row0 + t, sl])


def _lut_accum(tab_hbm, ridx_v, acc_v, b0_v, b1_v, sem0, sem1, ndet, width):
    bufs = (b0_v, b1_v)
    sems = (sem0, sem1)

    def start(pair, b):
        pltpu.async_copy(
            tab_hbm.at[ridx_v.at[pl.ds((1 + 2 * pair) * _TPW, 2 * _TPW)]],
            bufs[b], sems[b])

    def wait(b):
        pltpu.make_async_copy(tab_hbm.at[ridx_v.at[pl.ds(0, 2 * _TPW)]],
                              bufs[b], sems[b]).wait()

    npair = (ndet - 2) // 2
    nq = npair // 2

    pltpu.async_copy(tab_hbm.at[ridx_v.at[pl.ds(0, _TPW)]],
                     acc_v, sem0).wait()
    start(0, 0)
    start(1, 1)

    @pl.loop(0, nq)
    def _(q):
        p0 = 2 * q
        wait(0)
        _accum_rows(acc_v, b0_v, 0, width)
        _accum_rows(acc_v, b0_v, _TPW, width)

        @pl.when(p0 + 2 < npair)
        def _():
            start(p0 + 2, 0)

        wait(1)
        _accum_rows(acc_v, b1_v, 0, width)
        _accum_rows(acc_v, b1_v, _TPW, width)

        @pl.when(p0 + 3 < npair)
        def _():
            start(p0 + 3, 1)

    if npair % 2:
        wait(0)
        _accum_rows(acc_v, b0_v, 0, width)
        _accum_rows(acc_v, b0_v, _TPW, width)
    pltpu.async_copy(tab_hbm.at[ridx_v.at[pl.ds((ndet - 1) * _TPW, _TPW)]],
                     b1_v.at[pl.ds(0, _TPW)], sem1).wait()
    _accum_rows(acc_v, b1_v, 0, width)


def _residual_add(z_v, acc_v):
    @pl.loop(0, _TPW, unroll=2)
    def _(t):
        for c in range(_DIM // 16):
            sl = pl.ds(c * 16, 16)
            acc_v[t, sl] = z_v[t, sl] + acc_v[t, sl]


def _embed_body(tok_hbm, emb_hbm, z_hbm, tok_v, rows_v, sem):
    base = _wid() * _TPW
    pltpu.sync_copy(tok_hbm.at[pl.ds(base, _TPW)], tok_v)
    pltpu.async_copy(emb_hbm.at[tok_v], rows_v, sem).wait()
    pltpu.sync_copy(rows_v, z_hbm.at[pl.ds(base, _TPW)])


def _att_codes_body(z128_hbm, pos_hbm, aaT_hbm, atT_hbm, r_hbm,
                    aaT_v, atT_v, rowidx_v, vals_v, pos_v, out_v, sem):
    lanes = lax.iota(jnp.int32, 16)
    wid = _wid()
    base = wid * _TPW
    pltpu.sync_copy(aaT_hbm, aaT_v)
    pltpu.sync_copy(atT_hbm, atT_v)
    pltpu.sync_copy(pos_hbm.at[pl.ds(base, _TPW)], pos_v)

    accs = [jnp.zeros((16,), jnp.int32) for _ in range(_DET_ATT // 16)]
    for j in range(_ANCH):
        for g in range(_DET_ATT // 16):
            a = aaT_v[j, pl.ds(g * 16, 16)]
            row = jnp.minimum(lax.shift_right_logical(a, 7), _ZR128 - 1)
            rowidx_v[pl.ds(g * 16, 16)] = row
        pltpu.async_copy(z128_hbm.at[rowidx_v], vals_v, sem).wait()
        for g in range(_DET_ATT // 16):
            a = aaT_v[j, pl.ds(g * 16, 16)]
            th = atT_v[j, pl.ds(g * 16, 16)]
            val = plsc.load_gather(
                vals_v, [g * 16 + lanes, lax.bitwise_and(a, 127)])
            bit = jnp.logical_and(val > th, a < _SEQ)
            accs[g] = accs[g] + jnp.where(bit, 1 << j, 0)

    for g in range(_DET_ATT // 16):
        dvec = g * 16 + lanes
        rowbase = accs[g] + dvec * _POW2

        @pl.loop(0, _TPW)
        def _init(t, rowbase=rowbase, dvec=dvec):
            plsc.store_scatter(out_v, [dvec * _TPW + t], rowbase)

        for j in range(_ANCH):
            a = aaT_v[j, pl.ds(g * 16, 16)]
            th = atT_v[j, pl.ds(g * 16, 16)]
            ispos = a >= _SEQ
            col = jnp.clip(a - _SEQ, 0, _POS - 1)
            w = jnp.where(ispos, 1 << j, 0)

            @pl.when(jnp.max(ispos.astype(jnp.int32)) > 0)
            def _pos(col=col, th=th, w=w, dvec=dvec):
                @pl.loop(0, _TPW)
                def _(t):
                    tsp = jnp.full((16,), t, jnp.int32)
                    val = plsc.load_gather(pos_v, [tsp, col])
                    plsc.addupdate_scatter(out_v, [dvec * _TPW + t],
                                           jnp.where(val > th, w, 0))

    pltpu.sync_copy(out_v, r_hbm.at[wid])


def _row_codes_into(z_v, aT_v, tT_v, out_v, ndet):
    lanes = lax.iota(jnp.int32, 16)
    for tg in range(_TPW // 16):
        tvec = tg * 16 + lanes

        @pl.loop(0, ndet)
        def _(d, tvec=tvec):
            dsp = jnp.full((16,), d, jnp.int32)
            code = jnp.zeros((16,), jnp.int32)
            for j in range(_ANCH):
                jsp = jnp.full((16,), j, jnp.int32)
                a16 = plsc.load_gather(aT_v, [jsp, dsp])
                th16 = plsc.load_gather(tT_v, [jsp, dsp])
                val = plsc.load_gather(z_v, [tvec, a16])
                code = code + jnp.where(val > th16, 1 << j, 0)
            plsc.store_scatter(out_v, [dsp * _TPW + tvec],
                               code + d * _POW2)


def _att_sum_body(z_hbm, r_hbm, tab_hbm, faT_hbm, ftT_hbm,
                  zout_hbm, rffn_hbm,
                  z_v, acc_v, b0_v, b1_v, ridx_v, faT_v, ftT_v, rout_v,
                  sem0, sem1):
    wid = _wid()
    base = wid * _TPW
    pltpu.sync_copy(z_hbm.at[pl.ds(base, _TPW)], z_v)
    pltpu.sync_copy(r_hbm.at[wid], ridx_v)
    pltpu.sync_copy(faT_hbm, faT_v)
    pltpu.sync_copy(ftT_hbm, ftT_v)
    _lut_accum(tab_hbm, ridx_v, acc_v, b0_v, b1_v, sem0, sem1,
               _DET_ATT, _DIM)
    _residual_add(z_v, acc_v)
    pltpu.sync_copy(acc_v, zout_hbm.at[pl.ds(base, _TPW)])
    _row_codes_into(acc_v, faT_v, ftT_v, rout_v, _DET)
    pltpu.sync_copy(rout_v, rffn_hbm.at[wid])


def _ffn_sum_body(z_hbm, r_hbm, tab_hbm, zout_hbm,
                  z_v, acc_v, b0_v, b1_v, ridx_v, sem0, sem1):
    wid = _wid()
    base = wid * _TPW
    pltpu.sync_copy(z_hbm.at[pl.ds(base, _TPW)], z_v)
    pltpu.sync_copy(r_hbm.at[wid], ridx_v)
    _lut_accum(tab_hbm, ridx_v, acc_v, b0_v, b1_v, sem0, sem1, _DET, _DIM)
    _residual_add(z_v, acc_v)
    pltpu.sync_copy(acc_v, zout_hbm.at[pl.ds(base, _TPW)])


def _ffn_sum_codes_body(z_hbm, r_hbm, tab_hbm, uaT_hbm, utT_hbm,
                        zout_hbm, ru_hbm,
                        z_v, acc_v, b0_v, b1_v, ridx_v, uaT_v, utT_v, rout_v,
                        sem0, sem1):
    wid = _wid()
    base = wid * _TPW
    pltpu.sync_copy(z_hbm.at[pl.ds(base, _TPW)], z_v)
    pltpu.sync_copy(r_hbm.at[wid], ridx_v)
    pltpu.sync_copy(uaT_hbm, uaT_v)
    pltpu.sync_copy(utT_hbm, utT_v)
    _lut_accum(tab_hbm, ridx_v, acc_v, b0_v, b1_v, sem0, sem1, _DET, _DIM)
    _residual_add(z_v, acc_v)
    pltpu.sync_copy(acc_v, zout_hbm.at[pl.ds(base, _TPW)])
    _row_codes_into(acc_v, uaT_v, utT_v, rout_v, _DET)
    pltpu.sync_copy(rout_v, ru_hbm.at[wid])


def _tc_unemb_body(codes_ref, tab_ref, out_ref):
    codes = codes_ref[...]
    out_ref[...] = jnp.zeros_like(out_ref)
    for d in range(_DET):
        col = codes[:, d] - d * _POW2
        oh = (col[:, None] == lax.broadcasted_iota(
            jnp.int32, (codes.shape[0], _POW2), 1)).astype(jnp.bfloat16)
        out_ref[...] += lax.dot(
            oh, tab_ref[pl.ds(d * _POW2, _POW2), :],
            preferred_element_type=jnp.float32)


def kernel(tokens, token_emb_w, pos_enc,
           att_anchors_0, att_thresh_0, att_table_0,
           ffn_anchors_0, ffn_thresh_0, ffn_table_0,
           att_anchors_1, att_thresh_1, att_table_1,
           ffn_anchors_1, ffn_thresh_1, ffn_table_1,
           unemb_anchors, unemb_thresh, unemb_table):
    f32 = jnp.float32
    i32 = jnp.int32
    S = jax.ShapeDtypeStruct
    VM = pltpu.VMEM
    SEM = pltpu.SemaphoreType.DMA
    mesh = plsc.VectorSubcoreMesh(core_axis_name="c", subcore_axis_name="s")
    cp = pltpu.CompilerParams(needs_layout_passes=False)

    embed = pl.kernel(
        _embed_body, out_type=S((_CTX, _DIM), f32), mesh=mesh,
        compiler_params=cp,
        scratch_types=[VM((_TPW,), i32), VM((_TPW, _DIM), f32), SEM])

    att_codes = pl.kernel(
        _att_codes_body, out_type=S((_NW, _DET_ATT * _TPW), i32), mesh=mesh,
        compiler_params=cp,
        scratch_types=[VM((_ANCH, _DET_ATT), i32), VM((_ANCH, _DET_ATT), f32),
                       VM((_DET_ATT,), i32), VM((_DET_ATT, 128), f32),
                       VM((_TPW, _POS), f32), VM((_DET_ATT * _TPW,), i32), SEM])

    att_sum = pl.kernel(
        _att_sum_body,
        out_type=(S((_CTX, _DIM), f32), S((_NW, _DET * _TPW), i32)), mesh=mesh,
        compiler_params=cp,
        scratch_types=[VM((_TPW, _DIM), f32), VM((_TPW, _DIM), f32),
                       VM((2 * _TPW, _DIM), f32), VM((2 * _TPW, _DIM), f32),
                       VM((_DET_ATT * _TPW,), i32),
                       VM((_ANCH, _DET), i32), VM((_ANCH, _DET), f32),
                       VM((_DET * _TPW,), i32), SEM, SEM])

    ffn_sum = pl.kernel(
        _ffn_sum_body, out_type=S((_CTX, _DIM), f32), mesh=mesh,
        compiler_params=cp,
        scratch_types=[VM((_TPW, _DIM), f32), VM((_TPW, _DIM), f32),
                       VM((2 * _TPW, _DIM), f32), VM((2 * _TPW, _DIM), f32),
                       VM((_DET * _TPW,), i32), SEM, SEM])

    ffn_sum_codes = pl.kernel(
        _ffn_sum_codes_body,
        out_type=(S((_CTX, _DIM), f32), S((_NW, _DET * _TPW), i32)), mesh=mesh,
        compiler_params=cp,
        scratch_types=[VM((_TPW, _DIM), f32), VM((_TPW, _DIM), f32),
                       VM((2 * _TPW, _DIM), f32), VM((2 * _TPW, _DIM), f32),
                       VM((_DET * _TPW,), i32),
                       VM((_ANCH, _DET), i32), VM((_ANCH, _DET), f32),
                       VM((_DET * _TPW,), i32), SEM, SEM])

    TB = 256
    unemb_tc = pl.pallas_call(
        _tc_unemb_body,
        grid=(_CTX // TB,),
        in_specs=[pl.BlockSpec((TB, _DET), lambda i: (i, 0)),
                  pl.BlockSpec((_DET * _POW2, _VOCAB), lambda i: (0, 0))],
        out_specs=pl.BlockSpec((TB, _VOCAB), lambda i: (i, 0)),
        out_shape=S((_CTX, _VOCAB), f32))

    tok = tokens.reshape(_CTX)
    aaT = (att_anchors_0.T, att_anchors_1.T)
    atT = (att_thresh_0.T, att_thresh_1.T)
    faT = (ffn_anchors_0.T, ffn_anchors_1.T)
    ftT = (ffn_thresh_0.T, ffn_thresh_1.T)
    atab = (att_table_0.reshape(-1, _DIM), att_table_1.reshape(-1, _DIM))
    ftab = (ffn_table_0.reshape(-1, _DIM), ffn_table_1.reshape(-1, _DIM))
    utab_bf = unemb_table.reshape(-1, _VOCAB).astype(jnp.bfloat16)

    z = embed(tok, token_emb_w)
    ra0 = att_codes(z.reshape(_ZR128, 128), pos_enc, aaT[0], atT[0])
    z, rf0 = att_sum(z, ra0, atab[0], faT[0], ftT[0])
    z = ffn_sum(z, rf0, ftab[0])
    ra1 = att_codes(z.reshape(_ZR128, 128), pos_enc, aaT[1], atT[1])
    z, rf1 = att_sum(z, ra1, atab[1], faT[1], ftT[1])
    z, ru = ffn_sum_codes(z, rf1, ftab[1], unemb_anchors.T, unemb_thresh.T)
    ru_tok = ru.reshape(_NW, _DET, _TPW).transpose(0, 2, 1).reshape(_CTX, _DET)
    logits = unemb_tc(ru_tok, utab_bf)
    return logits.reshape(1, _CTX, _VOCAB)


# speedup vs baseline: 1.9898x; 1.1643x over previous
import jax
import jax.numpy as jnp
from jax import lax
from jax.experimental import pallas as pl
from jax.experimental.pallas import tpu as pltpu
from jax.experimental.pallas import tpu_sc as plsc

_VOCAB = 1000
_DIM = 256
_CTX = 2048
_POS = 16
_DET = 64
_ANCH = 8
_DET_ATT = 128
_SEQ = _CTX * _DIM
_POW2 = 256
_NW = 32
_TPW = _CTX // _NW
_ZR128 = _SEQ // 128
_VPAD = 1024


def _wid():
    return lax.axis_index("c") * 16 + lax.axis_index("s")


def _accum_rows(acc_v, buf_v, row0, width):
    @plsc.parallel_loop(0, _TPW, unroll=2)
    def _(t):
        for c in range(width // 16):
            sl = pl.ds(c * 16, 16)
            plsc.addupdate(acc_v.at[t, sl], buf_v[row0 + t, sl])


def _lut_accum(tab_hbm, ridx_v, acc_v, b0_v, b1_v, sem0, sem1, ndet, width):
    bufs = (b0_v, b1_v)
    sems = (sem0, sem1)

    def start(pair, b):
        pltpu.async_copy(
            tab_hbm.at[ridx_v.at[pl.ds((1 + 2 * pair) * _TPW, 2 * _TPW)]],
            bufs[b], sems[b])

    def wait(b):
        pltpu.make_async_copy(tab_hbm.at[ridx_v.at[pl.ds(0, 2 * _TPW)]],
                              bufs[b], sems[b]).wait()

    npair = (ndet - 2) // 2
    nq = npair // 2

    pltpu.async_copy(tab_hbm.at[ridx_v.at[pl.ds(0, _TPW)]],
                     acc_v, sem0).wait()
    start(0, 0)
    start(1, 1)

    @pl.loop(0, nq)
    def _(q):
        p0 = 2 * q
        wait(0)
        _accum_rows(acc_v, b0_v, 0, width)
        _accum_rows(acc_v, b0_v, _TPW, width)

        @pl.when(p0 + 2 < npair)
        def _():
            start(p0 + 2, 0)

        wait(1)
        _accum_rows(acc_v, b1_v, 0, width)
        _accum_rows(acc_v, b1_v, _TPW, width)

        @pl.when(p0 + 3 < npair)
        def _():
            start(p0 + 3, 1)

    if npair % 2:
        wait(0)
        _accum_rows(acc_v, b0_v, 0, width)
        _accum_rows(acc_v, b0_v, _TPW, width)
    pltpu.async_copy(tab_hbm.at[ridx_v.at[pl.ds((ndet - 1) * _TPW, _TPW)]],
                     b1_v.at[pl.ds(0, _TPW)], sem1).wait()
    _accum_rows(acc_v, b1_v, 0, width)


def _residual_add(z_v, acc_v):
    @plsc.parallel_loop(0, _TPW, unroll=2)
    def _(t):
        for c in range(_DIM // 16):
            sl = pl.ds(c * 16, 16)
            acc_v[t, sl] = z_v[t, sl] + acc_v[t, sl]


def _embed_body(tok_hbm, emb_hbm, z_hbm, tok_v, rows_v, sem):
    base = _wid() * _TPW
    pltpu.sync_copy(tok_hbm.at[pl.ds(base, _TPW)], tok_v)
    pltpu.async_copy(emb_hbm.at[tok_v], rows_v, sem).wait()
    pltpu.sync_copy(rows_v, z_hbm.at[pl.ds(base, _TPW)])


def _att_codes_body(z128_hbm, pos_hbm, aaT_hbm, atT_hbm, r_hbm,
                    aaT_v, atT_v, rowidx_v, vals_v, pos_v, out_v, sem):
    lanes = lax.iota(jnp.int32, 16)
    wid = _wid()
    base = wid * _TPW
    pltpu.sync_copy(aaT_hbm, aaT_v)
    pltpu.sync_copy(atT_hbm, atT_v)
    pltpu.sync_copy(pos_hbm.at[pl.ds(base, _TPW)], pos_v)

    accs = [jnp.zeros((16,), jnp.int32) for _ in range(_DET_ATT // 16)]
    for j in range(_ANCH):
        for g in range(_DET_ATT // 16):
            a = aaT_v[j, pl.ds(g * 16, 16)]
            row = jnp.minimum(lax.shift_right_logical(a, 7), _ZR128 - 1)
            rowidx_v[pl.ds(g * 16, 16)] = row
        pltpu.async_copy(z128_hbm.at[rowidx_v], vals_v, sem).wait()
        for g in range(_DET_ATT // 16):
            a = aaT_v[j, pl.ds(g * 16, 16)]
            th = atT_v[j, pl.ds(g * 16, 16)]
            val = plsc.load_gather(
                vals_v, [g * 16 + lanes, lax.bitwise_and(a, 127)])
            bit = jnp.logical_and(val > th, a < _SEQ)
            accs[g] = accs[g] + jnp.where(bit, 1 << j, 0)

    for g in range(_DET_ATT // 16):
        dvec = g * 16 + lanes
        rowbase = accs[g] + dvec * _POW2

        @pl.loop(0, _TPW)
        def _init(t, rowbase=rowbase, dvec=dvec):
            plsc.store_scatter(out_v, [dvec * _TPW + t], rowbase)

        for j in range(_ANCH):
            a = aaT_v[j, pl.ds(g * 16, 16)]
            th = atT_v[j, pl.ds(g * 16, 16)]
            ispos = a >= _SEQ
            col = jnp.clip(a - _SEQ, 0, _POS - 1)
            w = jnp.where(ispos, 1 << j, 0)

            @pl.when(jnp.max(ispos.astype(jnp.int32)) > 0)
            def _pos(col=col, th=th, w=w, dvec=dvec):
                @pl.loop(0, _TPW)
                def _(t):
                    tsp = jnp.full((16,), t, jnp.int32)
                    val = plsc.load_gather(pos_v, [tsp, col])
                    plsc.addupdate_scatter(out_v, [dvec * _TPW + t],
                                           jnp.where(val > th, w, 0))

    pltpu.sync_copy(out_v, r_hbm.at[wid])


def _row_codes_into(z_v, aT_v, tT_v, out_v, ndet):
    lanes = lax.iota(jnp.int32, 16)
    for tg in range(_TPW // 16):
        tvec = tg * 16 + lanes

        @pl.loop(0, ndet)
        def _(d, tvec=tvec):
            dsp = jnp.full((16,), d, jnp.int32)
            code = jnp.zeros((16,), jnp.int32)
            for j in range(_ANCH):
                jsp = jnp.full((16,), j, jnp.int32)
                a16 = plsc.load_gather(aT_v, [jsp, dsp])
                th16 = plsc.load_gather(tT_v, [jsp, dsp])
                val = plsc.load_gather(z_v, [tvec, a16])
                code = code + jnp.where(val > th16, 1 << j, 0)
            plsc.store_scatter(out_v, [dsp * _TPW + tvec],
                               code + d * _POW2)


def _att_sum_body(z_hbm, r_hbm, tab_hbm, faT_hbm, ftT_hbm,
                  zout_hbm, rffn_hbm,
                  z_v, acc_v, b0_v, b1_v, ridx_v, faT_v, ftT_v, rout_v,
                  sem0, sem1):
    wid = _wid()
    base = wid * _TPW
    pltpu.sync_copy(z_hbm.at[pl.ds(base, _TPW)], z_v)
    pltpu.sync_copy(r_hbm.at[wid], ridx_v)
    pltpu.sync_copy(faT_hbm, faT_v)
    pltpu.sync_copy(ftT_hbm, ftT_v)
    _lut_accum(tab_hbm, ridx_v, acc_v, b0_v, b1_v, sem0, sem1,
               _DET_ATT, _DIM)
    _residual_add(z_v, acc_v)
    pltpu.sync_copy(acc_v, zout_hbm.at[pl.ds(base, _TPW)])
    _row_codes_into(acc_v, faT_v, ftT_v, rout_v, _DET)
    pltpu.sync_copy(rout_v, rffn_hbm.at[wid])


def _ffn_sum_body(z_hbm, r_hbm, tab_hbm, zout_hbm,
                  z_v, acc_v, b0_v, b1_v, ridx_v, sem0, sem1):
    wid = _wid()
    base = wid * _TPW
    pltpu.sync_copy(z_hbm.at[pl.ds(base, _TPW)], z_v)
    pltpu.sync_copy(r_hbm.at[wid], ridx_v)
    _lut_accum(tab_hbm, ridx_v, acc_v, b0_v, b1_v, sem0, sem1, _DET, _DIM)
    _residual_add(z_v, acc_v)
    pltpu.sync_copy(acc_v, zout_hbm.at[pl.ds(base, _TPW)])


def _ffn_sum_codes_body(z_hbm, r_hbm, tab_hbm, uaT_hbm, utT_hbm,
                        zout_hbm, ru_hbm,
                        z_v, acc_v, b0_v, b1_v, ridx_v, uaT_v, utT_v, rout_v,
                        sem0, sem1):
    wid = _wid()
    base = wid * _TPW
    pltpu.sync_copy(z_hbm.at[pl.ds(base, _TPW)], z_v)
    pltpu.sync_copy(r_hbm.at[wid], ridx_v)
    pltpu.sync_copy(uaT_hbm, uaT_v)
    pltpu.sync_copy(utT_hbm, utT_v)
    _lut_accum(tab_hbm, ridx_v, acc_v, b0_v, b1_v, sem0, sem1, _DET, _DIM)
    _residual_add(z_v, acc_v)
    pltpu.sync_copy(acc_v, zout_hbm.at[pl.ds(base, _TPW)])
    _row_codes_into(acc_v, uaT_v, utT_v, rout_v, _DET)
    pltpu.sync_copy(rout_v, ru_hbm.at[wid])


def _tc_unemb_body(codes_ref, tab_ref, out_ref):
    codes = codes_ref[...]
    out_ref[...] = jnp.zeros_like(out_ref)
    for d in range(_DET):
        col = codes[:, d] - d * _POW2
        oh = (col[:, None] == lax.broadcasted_iota(
            jnp.int32, (codes.shape[0], _POW2), 1)).astype(jnp.bfloat16)
        out_ref[...] += lax.dot(
            oh, tab_ref[pl.ds(d * _POW2, _POW2), :],
            preferred_element_type=jnp.float32)


def kernel(tokens, token_emb_w, pos_enc,
           att_anchors_0, att_thresh_0, att_table_0,
           ffn_anchors_0, ffn_thresh_0, ffn_table_0,
           att_anchors_1, att_thresh_1, att_table_1,
           ffn_anchors_1, ffn_thresh_1, ffn_table_1,
           unemb_anchors, unemb_thresh, unemb_table):
    f32 = jnp.float32
    i32 = jnp.int32
    S = jax.ShapeDtypeStruct
    VM = pltpu.VMEM
    SEM = pltpu.SemaphoreType.DMA
    mesh = plsc.VectorSubcoreMesh(core_axis_name="c", subcore_axis_name="s")
    cp = pltpu.CompilerParams(needs_layout_passes=False)

    embed = pl.kernel(
        _embed_body, out_type=S((_CTX, _DIM), f32), mesh=mesh,
        compiler_params=cp,
        scratch_types=[VM((_TPW,), i32), VM((_TPW, _DIM), f32), SEM])

    att_codes = pl.kernel(
        _att_codes_body, out_type=S((_NW, _DET_ATT * _TPW), i32), mesh=mesh,
        compiler_params=cp,
        scratch_types=[VM((_ANCH, _DET_ATT), i32), VM((_ANCH, _DET_ATT), f32),
                       VM((_DET_ATT,), i32), VM((_DET_ATT, 128), f32),
                       VM((_TPW, _POS), f32), VM((_DET_ATT * _TPW,), i32), SEM])

    att_sum = pl.kernel(
        _att_sum_body,
        out_type=(S((_CTX, _DIM), f32), S((_NW, _DET * _TPW), i32)), mesh=mesh,
        compiler_params=cp,
        scratch_types=[VM((_TPW, _DIM), f32), VM((_TPW, _DIM), f32),
                       VM((2 * _TPW, _DIM), f32), VM((2 * _TPW, _DIM), f32),
                       VM((_DET_ATT * _TPW,), i32),
                       VM((_ANCH, _DET), i32), VM((_ANCH, _DET), f32),
                       VM((_DET * _TPW,), i32), SEM, SEM])

    ffn_sum = pl.kernel(
        _ffn_sum_body, out_type=S((_CTX, _DIM), f32), mesh=mesh,
        compiler_params=cp,
        scratch_types=[VM((_TPW, _DIM), f32), VM((_TPW, _DIM), f32),
                       VM((2 * _TPW, _DIM), f32), VM((2 * _TPW, _DIM), f32),
                       VM((_DET * _TPW,), i32), SEM, SEM])

    ffn_sum_codes = pl.kernel(
        _ffn_sum_codes_body,
        out_type=(S((_CTX, _DIM), f32), S((_NW, _DET * _TPW), i32)), mesh=mesh,
        compiler_params=cp,
        scratch_types=[VM((_TPW, _DIM), f32), VM((_TPW, _DIM), f32),
                       VM((2 * _TPW, _DIM), f32), VM((2 * _TPW, _DIM), f32),
                       VM((_DET * _TPW,), i32),
                       VM((_ANCH, _DET), i32), VM((_ANCH, _DET), f32),
                       VM((_DET * _TPW,), i32), SEM, SEM])

    TB = 256
    unemb_tc = pl.pallas_call(
        _tc_unemb_body,
        grid=(_CTX // TB,),
        in_specs=[pl.BlockSpec((TB, _DET), lambda i: (i, 0)),
                  pl.BlockSpec((_DET * _POW2, _VOCAB), lambda i: (0, 0))],
        out_specs=pl.BlockSpec((TB, _VOCAB), lambda i: (i, 0)),
        out_shape=S((_CTX, _VOCAB), f32))

    tok = tokens.reshape(_CTX)
    aaT = (att_anchors_0.T, att_anchors_1.T)
    atT = (att_thresh_0.T, att_thresh_1.T)
    faT = (ffn_anchors_0.T, ffn_anchors_1.T)
    ftT = (ffn_thresh_0.T, ffn_thresh_1.T)
    atab = (att_table_0.reshape(-1, _DIM), att_table_1.reshape(-1, _DIM))
    ftab = (ffn_table_0.reshape(-1, _DIM), ffn_table_1.reshape(-1, _DIM))
    utab_bf = unemb_table.reshape(-1, _VOCAB).astype(jnp.bfloat16)

    z = embed(tok, token_emb_w)
    ra0 = att_codes(z.reshape(_ZR128, 128), pos_enc, aaT[0], atT[0])
    z, rf0 = att_sum(z, ra0, atab[0], faT[0], ftT[0])
    z = ffn_sum(z, rf0, ftab[0])
    ra1 = att_codes(z.reshape(_ZR128, 128), pos_enc, aaT[1], atT[1])
    z, rf1 = att_sum(z, ra1, atab[1], faT[1], ftT[1])
    z, ru = ffn_sum_codes(z, rf1, ftab[1], unemb_anchors.T, unemb_thresh.T)
    ru_tok = ru.reshape(_NW, _DET, _TPW).transpose(0, 2, 1).reshape(_CTX, _DET)
    logits = unemb_tc(ru_tok, utab_bf)
    return logits.reshape(1, _CTX, _VOCAB)


# speedup vs baseline: 1.9926x; 1.0014x over previous
import jax
import jax.numpy as jnp
from jax import lax
from jax.experimental import pallas as pl
from jax.experimental.pallas import tpu as pltpu
from jax.experimental.pallas import tpu_sc as plsc

_VOCAB = 1000
_DIM = 256
_CTX = 2048
_POS = 16
_DET = 64
_ANCH = 8
_DET_ATT = 128
_SEQ = _CTX * _DIM
_POW2 = 256
_NW = 32
_TPW = _CTX // _NW
_ZR128 = _SEQ // 128
_VPAD = 1024


def _wid():
    return lax.axis_index("c") * 16 + lax.axis_index("s")


def _accum_rows(acc_v, buf_v, row0, width):
    @plsc.parallel_loop(0, _TPW, unroll=4)
    def _(t):
        for c in range(width // 16):
            sl = pl.ds(c * 16, 16)
            plsc.addupdate(acc_v.at[t, sl], buf_v[row0 + t, sl])


def _lut_accum(tab_hbm, ridx_v, acc_v, b0_v, b1_v, sem0, sem1, ndet, width):
    bufs = (b0_v, b1_v)
    sems = (sem0, sem1)

    def start(pair, b):
        pltpu.async_copy(
            tab_hbm.at[ridx_v.at[pl.ds((1 + 2 * pair) * _TPW, 2 * _TPW)]],
            bufs[b], sems[b])

    def wait(b):
        pltpu.make_async_copy(tab_hbm.at[ridx_v.at[pl.ds(0, 2 * _TPW)]],
                              bufs[b], sems[b]).wait()

    npair = (ndet - 2) // 2
    nq = npair // 2

    pltpu.async_copy(tab_hbm.at[ridx_v.at[pl.ds(0, _TPW)]],
                     acc_v, sem0).wait()
    start(0, 0)
    start(1, 1)

    @pl.loop(0, nq)
    def _(q):
        p0 = 2 * q
        wait(0)
        _accum_rows(acc_v, b0_v, 0, width)
        _accum_rows(acc_v, b0_v, _TPW, width)

        @pl.when(p0 + 2 < npair)
        def _():
            start(p0 + 2, 0)

        wait(1)
        _accum_rows(acc_v, b1_v, 0, width)
        _accum_rows(acc_v, b1_v, _TPW, width)

        @pl.when(p0 + 3 < npair)
        def _():
            start(p0 + 3, 1)

    if npair % 2:
        wait(0)
        _accum_rows(acc_v, b0_v, 0, width)
        _accum_rows(acc_v, b0_v, _TPW, width)
    pltpu.async_copy(tab_hbm.at[ridx_v.at[pl.ds((ndet - 1) * _TPW, _TPW)]],
                     b1_v.at[pl.ds(0, _TPW)], sem1).wait()
    _accum_rows(acc_v, b1_v, 0, width)


def _residual_add(z_v, acc_v):
    @plsc.parallel_loop(0, _TPW, unroll=2)
    def _(t):
        for c in range(_DIM // 16):
            sl = pl.ds(c * 16, 16)
            acc_v[t, sl] = z_v[t, sl] + acc_v[t, sl]


def _embed_body(tok_hbm, emb_hbm, z_hbm, tok_v, rows_v, sem):
    base = _wid() * _TPW
    pltpu.sync_copy(tok_hbm.at[pl.ds(base, _TPW)], tok_v)
    pltpu.async_copy(emb_hbm.at[tok_v], rows_v, sem).wait()
    pltpu.sync_copy(rows_v, z_hbm.at[pl.ds(base, _TPW)])


def _att_codes_body(z128_hbm, pos_hbm, aaT_hbm, atT_hbm, r_hbm,
                    aaT_v, atT_v, rowidx_v, vals_v, pos_v, out_v, sem):
    lanes = lax.iota(jnp.int32, 16)
    wid = _wid()
    base = wid * _TPW
    pltpu.sync_copy(aaT_hbm, aaT_v)
    pltpu.sync_copy(atT_hbm, atT_v)
    pltpu.sync_copy(pos_hbm.at[pl.ds(base, _TPW)], pos_v)

    accs = [jnp.zeros((16,), jnp.int32) for _ in range(_DET_ATT // 16)]
    for j in range(_ANCH):
        for g in range(_DET_ATT // 16):
            a = aaT_v[j, pl.ds(g * 16, 16)]
            row = jnp.minimum(lax.shift_right_logical(a, 7), _ZR128 - 1)
            rowidx_v[pl.ds(g * 16, 16)] = row
        pltpu.async_copy(z128_hbm.at[rowidx_v], vals_v, sem).wait()
        for g in range(_DET_ATT // 16):
            a = aaT_v[j, pl.ds(g * 16, 16)]
            th = atT_v[j, pl.ds(g * 16, 16)]
            val = plsc.load_gather(
                vals_v, [g * 16 + lanes, lax.bitwise_and(a, 127)])
            bit = jnp.logical_and(val > th, a < _SEQ)
            accs[g] = accs[g] + jnp.where(bit, 1 << j, 0)

    for g in range(_DET_ATT // 16):
        dvec = g * 16 + lanes
        rowbase = accs[g] + dvec * _POW2

        @plsc.parallel_loop(0, _TPW, unroll=2)
        def _init(t, rowbase=rowbase, dvec=dvec):
            plsc.store_scatter(out_v, [dvec * _TPW + t], rowbase)

        for j in range(_ANCH):
            a = aaT_v[j, pl.ds(g * 16, 16)]
            th = atT_v[j, pl.ds(g * 16, 16)]
            ispos = a >= _SEQ
            col = jnp.clip(a - _SEQ, 0, _POS - 1)
            w = jnp.where(ispos, 1 << j, 0)

            @pl.when(jnp.max(ispos.astype(jnp.int32)) > 0)
            def _pos(col=col, th=th, w=w, dvec=dvec):
                @plsc.parallel_loop(0, _TPW)
                def _(t):
                    tsp = jnp.full((16,), t, jnp.int32)
                    val = plsc.load_gather(pos_v, [tsp, col])
                    plsc.addupdate_scatter(out_v, [dvec * _TPW + t],
                                           jnp.where(val > th, w, 0))

    pltpu.sync_copy(out_v, r_hbm.at[wid])


def _row_codes_into(z_v, aT_v, tT_v, out_v, ndet):
    lanes = lax.iota(jnp.int32, 16)
    for tg in range(_TPW // 16):
        tvec = tg * 16 + lanes

        @plsc.parallel_loop(0, ndet, unroll=2)
        def _(d, tvec=tvec):
            dsp = jnp.full((16,), d, jnp.int32)
            code = jnp.zeros((16,), jnp.int32)
            for j in range(_ANCH):
                jsp = jnp.full((16,), j, jnp.int32)
                a16 = plsc.load_gather(aT_v, [jsp, dsp])
                th16 = plsc.load_gather(tT_v, [jsp, dsp])
                val = plsc.load_gather(z_v, [tvec, a16])
                code = code + jnp.where(val > th16, 1 << j, 0)
            plsc.store_scatter(out_v, [dsp * _TPW + tvec],
                               code + d * _POW2)


def _att_sum_body(z_hbm, r_hbm, tab_hbm, faT_hbm, ftT_hbm,
                  zout_hbm, rffn_hbm,
                  z_v, acc_v, b0_v, b1_v, ridx_v, faT_v, ftT_v, rout_v,
                  sem0, sem1):
    wid = _wid()
    base = wid * _TPW
    pltpu.sync_copy(z_hbm.at[pl.ds(base, _TPW)], z_v)
    pltpu.sync_copy(r_hbm.at[wid], ridx_v)
    pltpu.sync_copy(faT_hbm, faT_v)
    pltpu.sync_copy(ftT_hbm, ftT_v)
    _lut_accum(tab_hbm, ridx_v, acc_v, b0_v, b1_v, sem0, sem1,
               _DET_ATT, _DIM)
    _residual_add(z_v, acc_v)
    pltpu.sync_copy(acc_v, zout_hbm.at[pl.ds(base, _TPW)])
    _row_codes_into(acc_v, faT_v, ftT_v, rout_v, _DET)
    pltpu.sync_copy(rout_v, rffn_hbm.at[wid])


def _ffn_sum_body(z_hbm, r_hbm, tab_hbm, zout_hbm,
                  z_v, acc_v, b0_v, b1_v, ridx_v, sem0, sem1):
    wid = _wid()
    base = wid * _TPW
    pltpu.sync_copy(z_hbm.at[pl.ds(base, _TPW)], z_v)
    pltpu.sync_copy(r_hbm.at[wid], ridx_v)
    _lut_accum(tab_hbm, ridx_v, acc_v, b0_v, b1_v, sem0, sem1, _DET, _DIM)
    _residual_add(z_v, acc_v)
    pltpu.sync_copy(acc_v, zout_hbm.at[pl.ds(base, _TPW)])


def _ffn_sum_codes_body(z_hbm, r_hbm, tab_hbm, uaT_hbm, utT_hbm,
                        zout_hbm, ru_hbm,
                        z_v, acc_v, b0_v, b1_v, ridx_v, uaT_v, utT_v, rout_v,
                        sem0, sem1):
    wid = _wid()
    base = wid * _TPW
    pltpu.sync_copy(z_hbm.at[pl.ds(base, _TPW)], z_v)
    pltpu.sync_copy(r_hbm.at[wid], ridx_v)
    pltpu.sync_copy(uaT_hbm, uaT_v)
    pltpu.sync_copy(utT_hbm, utT_v)
    _lut_accum(tab_hbm, ridx_v, acc_v, b0_v, b1_v, sem0, sem1, _DET, _DIM)
    _residual_add(z_v, acc_v)
    pltpu.sync_copy(acc_v, zout_hbm.at[pl.ds(base, _TPW)])
    _row_codes_into(acc_v, uaT_v, utT_v, rout_v, _DET)
    pltpu.sync_copy(rout_v, ru_hbm.at[wid])


def _tc_unemb_body(codes_ref, tab_ref, out_ref):
    codes = codes_ref[...]
    out_ref[...] = jnp.zeros_like(out_ref)
    for d in range(_DET):
        col = codes[:, d] - d * _POW2
        oh = (col[:, None] == lax.broadcasted_iota(
            jnp.int32, (codes.shape[0], _POW2), 1)).astype(jnp.bfloat16)
        out_ref[...] += lax.dot(
            oh, tab_ref[pl.ds(d * _POW2, _POW2), :],
            preferred_element_type=jnp.float32)


def kernel(tokens, token_emb_w, pos_enc,
           att_anchors_0, att_thresh_0, att_table_0,
           ffn_anchors_0, ffn_thresh_0, ffn_table_0,
           att_anchors_1, att_thresh_1, att_table_1,
           ffn_anchors_1, ffn_thresh_1, ffn_table_1,
           unemb_anchors, unemb_thresh, unemb_table):
    f32 = jnp.float32
    i32 = jnp.int32
    S = jax.ShapeDtypeStruct
    VM = pltpu.VMEM
    SEM = pltpu.SemaphoreType.DMA
    mesh = plsc.VectorSubcoreMesh(core_axis_name="c", subcore_axis_name="s")
    cp = pltpu.CompilerParams(needs_layout_passes=False)

    embed = pl.kernel(
        _embed_body, out_type=S((_CTX, _DIM), f32), mesh=mesh,
        compiler_params=cp,
        scratch_types=[VM((_TPW,), i32), VM((_TPW, _DIM), f32), SEM])

    att_codes = pl.kernel(
        _att_codes_body, out_type=S((_NW, _DET_ATT * _TPW), i32), mesh=mesh,
        compiler_params=cp,
        scratch_types=[VM((_ANCH, _DET_ATT), i32), VM((_ANCH, _DET_ATT), f32),
                       VM((_DET_ATT,), i32), VM((_DET_ATT, 128), f32),
                       VM((_TPW, _POS), f32), VM((_DET_ATT * _TPW,), i32), SEM])

    att_sum = pl.kernel(
        _att_sum_body,
        out_type=(S((_CTX, _DIM), f32), S((_NW, _DET * _TPW), i32)), mesh=mesh,
        compiler_params=cp,
        scratch_types=[VM((_TPW, _DIM), f32), VM((_TPW, _DIM), f32),
                       VM((2 * _TPW, _DIM), f32), VM((2 * _TPW, _DIM), f32),
                       VM((_DET_ATT * _TPW,), i32),
                       VM((_ANCH, _DET), i32), VM((_ANCH, _DET), f32),
                       VM((_DET * _TPW,), i32), SEM, SEM])

    ffn_sum = pl.kernel(
        _ffn_sum_body, out_type=S((_CTX, _DIM), f32), mesh=mesh,
        compiler_params=cp,
        scratch_types=[VM((_TPW, _DIM), f32), VM((_TPW, _DIM), f32),
                       VM((2 * _TPW, _DIM), f32), VM((2 * _TPW, _DIM), f32),
                       VM((_DET * _TPW,), i32), SEM, SEM])

    ffn_sum_codes = pl.kernel(
        _ffn_sum_codes_body,
        out_type=(S((_CTX, _DIM), f32), S((_NW, _DET * _TPW), i32)), mesh=mesh,
        compiler_params=cp,
        scratch_types=[VM((_TPW, _DIM), f32), VM((_TPW, _DIM), f32),
                       VM((2 * _TPW, _DIM), f32), VM((2 * _TPW, _DIM), f32),
                       VM((_DET * _TPW,), i32),
                       VM((_ANCH, _DET), i32), VM((_ANCH, _DET), f32),
                       VM((_DET * _TPW,), i32), SEM, SEM])

    TB = 256
    unemb_tc = pl.pallas_call(
        _tc_unemb_body,
        grid=(_CTX // TB,),
        in_specs=[pl.BlockSpec((TB, _DET), lambda i: (i, 0)),
                  pl.BlockSpec((_DET * _POW2, _VOCAB), lambda i: (0, 0))],
        out_specs=pl.BlockSpec((TB, _VOCAB), lambda i: (i, 0)),
        out_shape=S((_CTX, _VOCAB), f32))

    tok = tokens.reshape(_CTX)
    aaT = (att_anchors_0.T, att_anchors_1.T)
    atT = (att_thresh_0.T, att_thresh_1.T)
    faT = (ffn_anchors_0.T, ffn_anchors_1.T)
    ftT = (ffn_thresh_0.T, ffn_thresh_1.T)
    atab = (att_table_0.reshape(-1, _DIM), att_table_1.reshape(-1, _DIM))
    ftab = (ffn_table_0.reshape(-1, _DIM), ffn_table_1.reshape(-1, _DIM))
    utab_bf = unemb_table.reshape(-1, _VOCAB).astype(jnp.bfloat16)

    z = embed(tok, token_emb_w)
    ra0 = att_codes(z.reshape(_ZR128, 128), pos_enc, aaT[0], atT[0])
    z, rf0 = att_sum(z, ra0, atab[0], faT[0], ftT[0])
    z = ffn_sum(z, rf0, ftab[0])
    ra1 = att_codes(z.reshape(_ZR128, 128), pos_enc, aaT[1], atT[1])
    z, rf1 = att_sum(z, ra1, atab[1], faT[1], ftT[1])
    z, ru = ffn_sum_codes(z, rf1, ftab[1], unemb_anchors.T, unemb_thresh.T)
    ru_tok = ru.reshape(_NW, _DET, _TPW).transpose(0, 2, 1).reshape(_CTX, _DET)
    logits = unemb_tc(ru_tok, utab_bf)
    return logits.reshape(1, _CTX, _VOCAB)


# speedup vs baseline: 2.1508x; 1.0794x over previous
import jax
import jax.numpy as jnp
from jax import lax
from jax.experimental import pallas as pl
from jax.experimental.pallas import tpu as pltpu
from jax.experimental.pallas import tpu_sc as plsc

_VOCAB = 1000
_DIM = 256
_CTX = 2048
_POS = 16
_DET = 64
_ANCH = 8
_DET_ATT = 128
_SEQ = _CTX * _DIM
_POW2 = 256
_NW = 32
_TPW = _CTX // _NW
_ZR128 = _SEQ // 128
_VPAD = 1024


def _wid():
    return lax.axis_index("c") * 16 + lax.axis_index("s")


def _accum_rows(acc_v, buf_v, row0, width):
    @plsc.parallel_loop(0, _TPW, unroll=4)
    def _(t):
        for c in range(width // 16):
            sl = pl.ds(c * 16, 16)
            plsc.addupdate(acc_v.at[t, sl], buf_v[row0 + t, sl])


def _lut_accum(tab_hbm, ridx_v, acc_v, bufs, sems, ndet, width):
    nbuf = len(bufs)

    def start(d, b):
        pltpu.async_copy(tab_hbm.at[ridx_v.at[pl.ds(d * _TPW, _TPW)]],
                         bufs[b], sems[b])

    def wait(b):
        pltpu.make_async_copy(tab_hbm.at[ridx_v.at[pl.ds(0, _TPW)]],
                              bufs[b], sems[b]).wait()

    pltpu.async_copy(tab_hbm.at[ridx_v.at[pl.ds(0, _TPW)]],
                     acc_v, sems[0]).wait()
    for b in range(nbuf):
        start(1 + b, b)

    nrem = ndet - 1
    rounds = nrem // nbuf
    tail = nrem % nbuf

    @pl.loop(0, rounds)
    def _(q):
        for b in range(nbuf):
            d = 1 + q * nbuf + b
            wait(b)
            _accum_rows(acc_v, bufs[b], 0, width)

            @pl.when(d + nbuf < ndet)
            def _(d=d, b=b):
                start(d + nbuf, b)

    for k in range(tail):
        wait(k)
        _accum_rows(acc_v, bufs[k], 0, width)


def _residual_add(z_v, acc_v):
    @plsc.parallel_loop(0, _TPW, unroll=2)
    def _(t):
        for c in range(_DIM // 16):
            sl = pl.ds(c * 16, 16)
            acc_v[t, sl] = z_v[t, sl] + acc_v[t, sl]


def _embed_body(tok_hbm, emb_hbm, z_hbm, tok_v, rows_v, sem):
    base = _wid() * _TPW
    pltpu.sync_copy(tok_hbm.at[pl.ds(base, _TPW)], tok_v)
    pltpu.async_copy(emb_hbm.at[tok_v], rows_v, sem).wait()
    pltpu.sync_copy(rows_v, z_hbm.at[pl.ds(base, _TPW)])


def _att_codes_body(z128_hbm, pos_hbm, aaT_hbm, atT_hbm, r_hbm,
                    aaT_v, atT_v, rowidx_v, vals_v, pos_v, out_v, sem):
    lanes = lax.iota(jnp.int32, 16)
    wid = _wid()
    base = wid * _TPW
    pltpu.sync_copy(aaT_hbm, aaT_v)
    pltpu.sync_copy(atT_hbm, atT_v)
    pltpu.sync_copy(pos_hbm.at[pl.ds(base, _TPW)], pos_v)

    accs = [jnp.zeros((16,), jnp.int32) for _ in range(_DET_ATT // 16)]
    for j in range(_ANCH):
        for g in range(_DET_ATT // 16):
            a = aaT_v[j, pl.ds(g * 16, 16)]
            row = jnp.minimum(lax.shift_right_logical(a, 7), _ZR128 - 1)
            rowidx_v[pl.ds(g * 16, 16)] = row
        pltpu.async_copy(z128_hbm.at[rowidx_v], vals_v, sem).wait()
        for g in range(_DET_ATT // 16):
            a = aaT_v[j, pl.ds(g * 16, 16)]
            th = atT_v[j, pl.ds(g * 16, 16)]
            val = plsc.load_gather(
                vals_v, [g * 16 + lanes, lax.bitwise_and(a, 127)])
            bit = jnp.logical_and(val > th, a < _SEQ)
            accs[g] = accs[g] + jnp.where(bit, 1 << j, 0)

    for g in range(_DET_ATT // 16):
        dvec = g * 16 + lanes
        rowbase = accs[g] + dvec * _POW2

        @plsc.parallel_loop(0, _TPW, unroll=2)
        def _init(t, rowbase=rowbase, dvec=dvec):
            plsc.store_scatter(out_v, [dvec * _TPW + t], rowbase)

        for j in range(_ANCH):
            a = aaT_v[j, pl.ds(g * 16, 16)]
            th = atT_v[j, pl.ds(g * 16, 16)]
            ispos = a >= _SEQ
            col = jnp.clip(a - _SEQ, 0, _POS - 1)
            w = jnp.where(ispos, 1 << j, 0)

            @pl.when(jnp.max(ispos.astype(jnp.int32)) > 0)
            def _pos(col=col, th=th, w=w, dvec=dvec):
                @plsc.parallel_loop(0, _TPW)
                def _(t):
                    tsp = jnp.full((16,), t, jnp.int32)
                    val = plsc.load_gather(pos_v, [tsp, col])
                    plsc.addupdate_scatter(out_v, [dvec * _TPW + t],
                                           jnp.where(val > th, w, 0))

    pltpu.sync_copy(out_v, r_hbm.at[wid])


def _row_codes_into(z_v, aT_v, tT_v, out_v, ndet):
    lanes = lax.iota(jnp.int32, 16)
    for tg in range(_TPW // 16):
        tvec = tg * 16 + lanes

        @plsc.parallel_loop(0, ndet, unroll=2)
        def _(d, tvec=tvec):
            dsp = jnp.full((16,), d, jnp.int32)
            code = jnp.zeros((16,), jnp.int32)
            for j in range(_ANCH):
                jsp = jnp.full((16,), j, jnp.int32)
                a16 = plsc.load_gather(aT_v, [jsp, dsp])
                th16 = plsc.load_gather(tT_v, [jsp, dsp])
                val = plsc.load_gather(z_v, [tvec, a16])
                code = code + jnp.where(val > th16, 1 << j, 0)
            plsc.store_scatter(out_v, [dsp * _TPW + tvec],
                               code + d * _POW2)


def _att_sum_body(z_hbm, r_hbm, tab_hbm, faT_hbm, ftT_hbm,
                  zout_hbm, rffn_hbm,
                  z_v, acc_v, b0_v, b1_v, b2_v, b3_v, ridx_v,
                  faT_v, ftT_v, rout_v, sem0, sem1, sem2, sem3):
    wid = _wid()
    base = wid * _TPW
    pltpu.sync_copy(z_hbm.at[pl.ds(base, _TPW)], z_v)
    pltpu.sync_copy(r_hbm.at[wid], ridx_v)
    pltpu.sync_copy(faT_hbm, faT_v)
    pltpu.sync_copy(ftT_hbm, ftT_v)
    _lut_accum(tab_hbm, ridx_v, acc_v, (b0_v, b1_v, b2_v, b3_v),
               (sem0, sem1, sem2, sem3), _DET_ATT, _DIM)
    _residual_add(z_v, acc_v)
    pltpu.sync_copy(acc_v, zout_hbm.at[pl.ds(base, _TPW)])
    _row_codes_into(acc_v, faT_v, ftT_v, rout_v, _DET)
    pltpu.sync_copy(rout_v, rffn_hbm.at[wid])


def _ffn_sum_body(z_hbm, r_hbm, tab_hbm, zout_hbm,
                  z_v, acc_v, b0_v, b1_v, b2_v, b3_v, ridx_v,
                  sem0, sem1, sem2, sem3):
    wid = _wid()
    base = wid * _TPW
    pltpu.sync_copy(z_hbm.at[pl.ds(base, _TPW)], z_v)
    pltpu.sync_copy(r_hbm.at[wid], ridx_v)
    _lut_accum(tab_hbm, ridx_v, acc_v, (b0_v, b1_v, b2_v, b3_v),
               (sem0, sem1, sem2, sem3), _DET, _DIM)
    _residual_add(z_v, acc_v)
    pltpu.sync_copy(acc_v, zout_hbm.at[pl.ds(base, _TPW)])


def _ffn_sum_codes_body(z_hbm, r_hbm, tab_hbm, uaT_hbm, utT_hbm,
                        zout_hbm, ru_hbm,
                        z_v, acc_v, b0_v, b1_v, b2_v, b3_v, ridx_v,
                        uaT_v, utT_v, rout_v, sem0, sem1, sem2, sem3):
    wid = _wid()
    base = wid * _TPW
    pltpu.sync_copy(z_hbm.at[pl.ds(base, _TPW)], z_v)
    pltpu.sync_copy(r_hbm.at[wid], ridx_v)
    pltpu.sync_copy(uaT_hbm, uaT_v)
    pltpu.sync_copy(utT_hbm, utT_v)
    _lut_accum(tab_hbm, ridx_v, acc_v, (b0_v, b1_v, b2_v, b3_v),
               (sem0, sem1, sem2, sem3), _DET, _DIM)
    _residual_add(z_v, acc_v)
    pltpu.sync_copy(acc_v, zout_hbm.at[pl.ds(base, _TPW)])
    _row_codes_into(acc_v, uaT_v, utT_v, rout_v, _DET)
    pltpu.sync_copy(rout_v, ru_hbm.at[wid])


def _tc_unemb_body(codes_ref, tab_ref, out_ref):
    codes = codes_ref[...]
    out_ref[...] = jnp.zeros_like(out_ref)
    for d in range(_DET):
        col = codes[:, d] - d * _POW2
        oh = (col[:, None] == lax.broadcasted_iota(
            jnp.int32, (codes.shape[0], _POW2), 1)).astype(jnp.bfloat16)
        out_ref[...] += lax.dot(
            oh, tab_ref[pl.ds(d * _POW2, _POW2), :],
            preferred_element_type=jnp.float32)


def kernel(tokens, token_emb_w, pos_enc,
           att_anchors_0, att_thresh_0, att_table_0,
           ffn_anchors_0, ffn_thresh_0, ffn_table_0,
           att_anchors_1, att_thresh_1, att_table_1,
           ffn_anchors_1, ffn_thresh_1, ffn_table_1,
           unemb_anchors, unemb_thresh, unemb_table):
    f32 = jnp.float32
    i32 = jnp.int32
    S = jax.ShapeDtypeStruct
    VM = pltpu.VMEM
    SEM = pltpu.SemaphoreType.DMA
    mesh = plsc.VectorSubcoreMesh(core_axis_name="c", subcore_axis_name="s")
    cp = pltpu.CompilerParams(needs_layout_passes=False)

    embed = pl.kernel(
        _embed_body, out_type=S((_CTX, _DIM), f32), mesh=mesh,
        compiler_params=cp,
        scratch_types=[VM((_TPW,), i32), VM((_TPW, _DIM), f32), SEM])

    att_codes = pl.kernel(
        _att_codes_body, out_type=S((_NW, _DET_ATT * _TPW), i32), mesh=mesh,
        compiler_params=cp,
        scratch_types=[VM((_ANCH, _DET_ATT), i32), VM((_ANCH, _DET_ATT), f32),
                       VM((_DET_ATT,), i32), VM((_DET_ATT, 128), f32),
                       VM((_TPW, _POS), f32), VM((_DET_ATT * _TPW,), i32), SEM])

    att_sum = pl.kernel(
        _att_sum_body,
        out_type=(S((_CTX, _DIM), f32), S((_NW, _DET * _TPW), i32)), mesh=mesh,
        compiler_params=cp,
        scratch_types=[VM((_TPW, _DIM), f32), VM((_TPW, _DIM), f32),
                       VM((_TPW, _DIM), f32), VM((_TPW, _DIM), f32),
                       VM((_TPW, _DIM), f32), VM((_TPW, _DIM), f32),
                       VM((_DET_ATT * _TPW,), i32),
                       VM((_ANCH, _DET), i32), VM((_ANCH, _DET), f32),
                       VM((_DET * _TPW,), i32), SEM, SEM, SEM, SEM])

    ffn_sum = pl.kernel(
        _ffn_sum_body, out_type=S((_CTX, _DIM), f32), mesh=mesh,
        compiler_params=cp,
        scratch_types=[VM((_TPW, _DIM), f32), VM((_TPW, _DIM), f32),
                       VM((_TPW, _DIM), f32), VM((_TPW, _DIM), f32),
                       VM((_TPW, _DIM), f32), VM((_TPW, _DIM), f32),
                       VM((_DET * _TPW,), i32), SEM, SEM, SEM, SEM])

    ffn_sum_codes = pl.kernel(
        _ffn_sum_codes_body,
        out_type=(S((_CTX, _DIM), f32), S((_NW, _DET * _TPW), i32)), mesh=mesh,
        compiler_params=cp,
        scratch_types=[VM((_TPW, _DIM), f32), VM((_TPW, _DIM), f32),
                       VM((_TPW, _DIM), f32), VM((_TPW, _DIM), f32),
                       VM((_TPW, _DIM), f32), VM((_TPW, _DIM), f32),
                       VM((_DET * _TPW,), i32),
                       VM((_ANCH, _DET), i32), VM((_ANCH, _DET), f32),
                       VM((_DET * _TPW,), i32), SEM, SEM, SEM, SEM])

    TB = 256
    unemb_tc = pl.pallas_call(
        _tc_unemb_body,
        grid=(_CTX // TB,),
        in_specs=[pl.BlockSpec((TB, _DET), lambda i: (i, 0)),
                  pl.BlockSpec((_DET * _POW2, _VOCAB), lambda i: (0, 0))],
        out_specs=pl.BlockSpec((TB, _VOCAB), lambda i: (i, 0)),
        out_shape=S((_CTX, _VOCAB), f32))

    tok = tokens.reshape(_CTX)
    aaT = (att_anchors_0.T, att_anchors_1.T)
    atT = (att_thresh_0.T, att_thresh_1.T)
    faT = (ffn_anchors_0.T, ffn_anchors_1.T)
    ftT = (ffn_thresh_0.T, ffn_thresh_1.T)
    atab = (att_table_0.reshape(-1, _DIM), att_table_1.reshape(-1, _DIM))
    ftab = (ffn_table_0.reshape(-1, _DIM), ffn_table_1.reshape(-1, _DIM))
    utab_bf = unemb_table.reshape(-1, _VOCAB).astype(jnp.bfloat16)

    z = embed(tok, token_emb_w)
    ra0 = att_codes(z.reshape(_ZR128, 128), pos_enc, aaT[0], atT[0])
    z, rf0 = att_sum(z, ra0, atab[0], faT[0], ftT[0])
    z = ffn_sum(z, rf0, ftab[0])
    ra1 = att_codes(z.reshape(_ZR128, 128), pos_enc, aaT[1], atT[1])
    z, rf1 = att_sum(z, ra1, atab[1], faT[1], ftT[1])
    z, ru = ffn_sum_codes(z, rf1, ftab[1], unemb_anchors.T, unemb_thresh.T)
    ru_tok = ru.reshape(_NW, _DET, _TPW).transpose(0, 2, 1).reshape(_CTX, _DET)
    logits = unemb_tc(ru_tok, utab_bf)
    return logits.reshape(1, _CTX, _VOCAB)


# speedup vs baseline: 2.1698x; 1.0089x over previous
import jax
import jax.numpy as jnp
from jax import lax
from jax.experimental import pallas as pl
from jax.experimental.pallas import tpu as pltpu
from jax.experimental.pallas import tpu_sc as plsc

_VOCAB = 1000
_DIM = 256
_CTX = 2048
_POS = 16
_DET = 64
_ANCH = 8
_DET_ATT = 128
_SEQ = _CTX * _DIM
_POW2 = 256
_NW = 32
_TPW = _CTX // _NW
_ZR128 = _SEQ // 128
_VPAD = 1024


def _wid():
    return lax.axis_index("c") * 16 + lax.axis_index("s")


def _accum_rows(acc_v, buf_v, row0, width):
    @plsc.parallel_loop(0, _TPW, unroll=4)
    def _(t):
        for c in range(width // 16):
            sl = pl.ds(c * 16, 16)
            plsc.addupdate(acc_v.at[t, sl], buf_v[row0 + t, sl])


def _lut_accum(tab_hbm, ridx_v, acc_v, bufs, sems, ndet, width):
    nbuf = len(bufs)

    def start(d, b):
        pltpu.async_copy(tab_hbm.at[ridx_v.at[pl.ds(d * _TPW, _TPW)]],
                         bufs[b], sems[b])

    def wait(b):
        pltpu.make_async_copy(tab_hbm.at[ridx_v.at[pl.ds(0, _TPW)]],
                              bufs[b], sems[b]).wait()

    pltpu.async_copy(tab_hbm.at[ridx_v.at[pl.ds(0, _TPW)]],
                     acc_v, sems[0]).wait()
    for b in range(nbuf):
        start(1 + b, b)

    nrem = ndet - 1
    rounds = nrem // nbuf
    tail = nrem % nbuf

    @pl.loop(0, rounds)
    def _(q):
        for b in range(nbuf):
            d = 1 + q * nbuf + b
            wait(b)
            _accum_rows(acc_v, bufs[b], 0, width)

            @pl.when(d + nbuf < ndet)
            def _(d=d, b=b):
                start(d + nbuf, b)

    for k in range(tail):
        wait(k)
        _accum_rows(acc_v, bufs[k], 0, width)


def _residual_add(z_v, acc_v):
    @plsc.parallel_loop(0, _TPW, unroll=2)
    def _(t):
        for c in range(_DIM // 16):
            sl = pl.ds(c * 16, 16)
            acc_v[t, sl] = z_v[t, sl] + acc_v[t, sl]


def _embed_body(tok_hbm, emb_hbm, z_hbm, tok_v, rows_v, sem):
    base = _wid() * _TPW
    pltpu.sync_copy(tok_hbm.at[pl.ds(base, _TPW)], tok_v)
    pltpu.async_copy(emb_hbm.at[tok_v], rows_v, sem).wait()
    pltpu.sync_copy(rows_v, z_hbm.at[pl.ds(base, _TPW)])


def _att_codes_body(z128_hbm, pos_hbm, aaT_hbm, atT_hbm, r_hbm,
                    aaT_v, atT_v, rowidx_v, vals_v, pos_v, out_v, sem):
    lanes = lax.iota(jnp.int32, 16)
    wid = _wid()
    base = wid * _TPW
    pltpu.sync_copy(aaT_hbm, aaT_v)
    pltpu.sync_copy(atT_hbm, atT_v)
    pltpu.sync_copy(pos_hbm.at[pl.ds(base, _TPW)], pos_v)

    accs = [jnp.zeros((16,), jnp.int32) for _ in range(_DET_ATT // 16)]
    for j in range(_ANCH):
        for g in range(_DET_ATT // 16):
            a = aaT_v[j, pl.ds(g * 16, 16)]
            row = jnp.minimum(lax.shift_right_logical(a, 7), _ZR128 - 1)
            rowidx_v[pl.ds(g * 16, 16)] = row
        pltpu.async_copy(z128_hbm.at[rowidx_v], vals_v, sem).wait()
        for g in range(_DET_ATT // 16):
            a = aaT_v[j, pl.ds(g * 16, 16)]
            th = atT_v[j, pl.ds(g * 16, 16)]
            val = plsc.load_gather(
                vals_v, [g * 16 + lanes, lax.bitwise_and(a, 127)])
            bit = jnp.logical_and(val > th, a < _SEQ)
            accs[g] = accs[g] + jnp.where(bit, 1 << j, 0)

    for g in range(_DET_ATT // 16):
        dvec = g * 16 + lanes
        rowbase = accs[g] + dvec * _POW2

        @plsc.parallel_loop(0, _TPW, unroll=2)
        def _init(t, rowbase=rowbase, dvec=dvec):
            plsc.store_scatter(out_v, [dvec * _TPW + t], rowbase)

        for j in range(_ANCH):
            a = aaT_v[j, pl.ds(g * 16, 16)]
            th = atT_v[j, pl.ds(g * 16, 16)]
            ispos = a >= _SEQ
            col = jnp.clip(a - _SEQ, 0, _POS - 1)
            w = jnp.where(ispos, 1 << j, 0)

            @pl.when(jnp.max(ispos.astype(jnp.int32)) > 0)
            def _pos(col=col, th=th, w=w, dvec=dvec):
                @plsc.parallel_loop(0, _TPW)
                def _(t):
                    tsp = jnp.full((16,), t, jnp.int32)
                    val = plsc.load_gather(pos_v, [tsp, col])
                    plsc.addupdate_scatter(out_v, [dvec * _TPW + t],
                                           jnp.where(val > th, w, 0))

    pltpu.sync_copy(out_v, r_hbm.at[wid])


def _row_codes_into(z_v, aT_v, tT_v, out_v, ndet):
    lanes = lax.iota(jnp.int32, 16)
    for tg in range(_TPW // 16):
        tvec = tg * 16 + lanes

        @plsc.parallel_loop(0, ndet, unroll=2)
        def _(d, tvec=tvec):
            dsp = jnp.full((16,), d, jnp.int32)
            code = jnp.zeros((16,), jnp.int32)
            for j in range(_ANCH):
                jsp = jnp.full((16,), j, jnp.int32)
                a16 = plsc.load_gather(aT_v, [jsp, dsp])
                th16 = plsc.load_gather(tT_v, [jsp, dsp])
                val = plsc.load_gather(z_v, [tvec, a16])
                code = code + jnp.where(val > th16, 1 << j, 0)
            plsc.store_scatter(out_v, [dsp * _TPW + tvec],
                               code + d * _POW2)


def _att_sum_body(z_hbm, r_hbm, tab_hbm, faT_hbm, ftT_hbm,
                  zout_hbm, rffn_hbm,
                  z_v, acc_v, b0_v, b1_v, b2_v, b3_v, b4_v, ridx_v,
                  faT_v, ftT_v, rout_v, sem0, sem1, sem2, sem3, sem4):
    wid = _wid()
    base = wid * _TPW
    pltpu.sync_copy(z_hbm.at[pl.ds(base, _TPW)], z_v)
    pltpu.sync_copy(r_hbm.at[wid], ridx_v)
    pltpu.sync_copy(faT_hbm, faT_v)
    pltpu.sync_copy(ftT_hbm, ftT_v)
    _lut_accum(tab_hbm, ridx_v, acc_v, (b0_v, b1_v, b2_v, b3_v, b4_v),
               (sem0, sem1, sem2, sem3, sem4), _DET_ATT, _DIM)
    _residual_add(z_v, acc_v)
    pltpu.sync_copy(acc_v, zout_hbm.at[pl.ds(base, _TPW)])
    _row_codes_into(acc_v, faT_v, ftT_v, rout_v, _DET)
    pltpu.sync_copy(rout_v, rffn_hbm.at[wid])


def _ffn_sum_body(z_hbm, r_hbm, tab_hbm, zout_hbm,
                  z_v, acc_v, b0_v, b1_v, b2_v, b3_v, ridx_v,
                  sem0, sem1, sem2, sem3):
    wid = _wid()
    base = wid * _TPW
    pltpu.sync_copy(z_hbm.at[pl.ds(base, _TPW)], z_v)
    pltpu.sync_copy(r_hbm.at[wid], ridx_v)
    _lut_accum(tab_hbm, ridx_v, acc_v, (b0_v, b1_v, b2_v, b3_v),
               (sem0, sem1, sem2, sem3), _DET, _DIM)
    _residual_add(z_v, acc_v)
    pltpu.sync_copy(acc_v, zout_hbm.at[pl.ds(base, _TPW)])


def _ffn_sum_codes_body(z_hbm, r_hbm, tab_hbm, uaT_hbm, utT_hbm,
                        zout_hbm, ru_hbm,
                        z_v, acc_v, b0_v, b1_v, b2_v, b3_v, ridx_v,
                        uaT_v, utT_v, rout_v, sem0, sem1, sem2, sem3):
    wid = _wid()
    base = wid * _TPW
    pltpu.sync_copy(z_hbm.at[pl.ds(base, _TPW)], z_v)
    pltpu.sync_copy(r_hbm.at[wid], ridx_v)
    pltpu.sync_copy(uaT_hbm, uaT_v)
    pltpu.sync_copy(utT_hbm, utT_v)
    _lut_accum(tab_hbm, ridx_v, acc_v, (b0_v, b1_v, b2_v, b3_v),
               (sem0, sem1, sem2, sem3), _DET, _DIM)
    _residual_add(z_v, acc_v)
    pltpu.sync_copy(acc_v, zout_hbm.at[pl.ds(base, _TPW)])
    _row_codes_into(acc_v, uaT_v, utT_v, rout_v, _DET)
    pltpu.sync_copy(rout_v, ru_hbm.at[wid])


def _tc_unemb_body(codes_ref, tab_ref, out_ref):
    codes = codes_ref[...]
    out_ref[...] = jnp.zeros_like(out_ref)
    for d in range(_DET):
        col = codes[:, d] - d * _POW2
        oh = (col[:, None] == lax.broadcasted_iota(
            jnp.int32, (codes.shape[0], _POW2), 1)).astype(jnp.bfloat16)
        out_ref[...] += lax.dot(
            oh, tab_ref[pl.ds(d * _POW2, _POW2), :],
            preferred_element_type=jnp.float32)


def kernel(tokens, token_emb_w, pos_enc,
           att_anchors_0, att_thresh_0, att_table_0,
           ffn_anchors_0, ffn_thresh_0, ffn_table_0,
           att_anchors_1, att_thresh_1, att_table_1,
           ffn_anchors_1, ffn_thresh_1, ffn_table_1,
           unemb_anchors, unemb_thresh, unemb_table):
    f32 = jnp.float32
    i32 = jnp.int32
    S = jax.ShapeDtypeStruct
    VM = pltpu.VMEM
    SEM = pltpu.SemaphoreType.DMA
    mesh = plsc.VectorSubcoreMesh(core_axis_name="c", subcore_axis_name="s")
    cp = pltpu.CompilerParams(needs_layout_passes=False)

    embed = pl.kernel(
        _embed_body, out_type=S((_CTX, _DIM), f32), mesh=mesh,
        compiler_params=cp,
        scratch_types=[VM((_TPW,), i32), VM((_TPW, _DIM), f32), SEM])

    att_codes = pl.kernel(
        _att_codes_body, out_type=S((_NW, _DET_ATT * _TPW), i32), mesh=mesh,
        compiler_params=cp,
        scratch_types=[VM((_ANCH, _DET_ATT), i32), VM((_ANCH, _DET_ATT), f32),
                       VM((_DET_ATT,), i32), VM((_DET_ATT, 128), f32),
                       VM((_TPW, _POS), f32), VM((_DET_ATT * _TPW,), i32), SEM])

    att_sum = pl.kernel(
        _att_sum_body,
        out_type=(S((_CTX, _DIM), f32), S((_NW, _DET * _TPW), i32)), mesh=mesh,
        compiler_params=cp,
        scratch_types=[VM((_TPW, _DIM), f32), VM((_TPW, _DIM), f32),
                       VM((_TPW, _DIM), f32), VM((_TPW, _DIM), f32),
                       VM((_TPW, _DIM), f32), VM((_TPW, _DIM), f32),
                       VM((_TPW, _DIM), f32),
                       VM((_DET_ATT * _TPW,), i32),
                       VM((_ANCH, _DET), i32), VM((_ANCH, _DET), f32),
                       VM((_DET * _TPW,), i32), SEM, SEM, SEM, SEM, SEM])

    ffn_sum = pl.kernel(
        _ffn_sum_body, out_type=S((_CTX, _DIM), f32), mesh=mesh,
        compiler_params=cp,
        scratch_types=[VM((_TPW, _DIM), f32), VM((_TPW, _DIM), f32),
                       VM((_TPW, _DIM), f32), VM((_TPW, _DIM), f32),
                       VM((_TPW, _DIM), f32), VM((_TPW, _DIM), f32),
                       VM((_DET * _TPW,), i32), SEM, SEM, SEM, SEM])

    ffn_sum_codes = pl.kernel(
        _ffn_sum_codes_body,
        out_type=(S((_CTX, _DIM), f32), S((_NW, _DET * _TPW), i32)), mesh=mesh,
        compiler_params=cp,
        scratch_types=[VM((_TPW, _DIM), f32), VM((_TPW, _DIM), f32),
                       VM((_TPW, _DIM), f32), VM((_TPW, _DIM), f32),
                       VM((_TPW, _DIM), f32), VM((_TPW, _DIM), f32),
                       VM((_DET * _TPW,), i32),
                       VM((_ANCH, _DET), i32), VM((_ANCH, _DET), f32),
                       VM((_DET * _TPW,), i32), SEM, SEM, SEM, SEM])

    TB = 256
    unemb_tc = pl.pallas_call(
        _tc_unemb_body,
        grid=(_CTX // TB,),
        in_specs=[pl.BlockSpec((TB, _DET), lambda i: (i, 0)),
                  pl.BlockSpec((_DET * _POW2, _VOCAB), lambda i: (0, 0))],
        out_specs=pl.BlockSpec((TB, _VOCAB), lambda i: (i, 0)),
        out_shape=S((_CTX, _VOCAB), f32))

    tok = tokens.reshape(_CTX)
    aaT = (att_anchors_0.T, att_anchors_1.T)
    atT = (att_thresh_0.T, att_thresh_1.T)
    faT = (ffn_anchors_0.T, ffn_anchors_1.T)
    ftT = (ffn_thresh_0.T, ffn_thresh_1.T)
    atab = (att_table_0.reshape(-1, _DIM), att_table_1.reshape(-1, _DIM))
    ftab = (ffn_table_0.reshape(-1, _DIM), ffn_table_1.reshape(-1, _DIM))
    utab_bf = unemb_table.reshape(-1, _VOCAB).astype(jnp.bfloat16)

    z = embed(tok, token_emb_w)
    ra0 = att_codes(z.reshape(_ZR128, 128), pos_enc, aaT[0], atT[0])
    z, rf0 = att_sum(z, ra0, atab[0], faT[0], ftT[0])
    z = ffn_sum(z, rf0, ftab[0])
    ra1 = att_codes(z.reshape(_ZR128, 128), pos_enc, aaT[1], atT[1])
    z, rf1 = att_sum(z, ra1, atab[1], faT[1], ftT[1])
    z, ru = ffn_sum_codes(z, rf1, ftab[1], unemb_anchors.T, unemb_thresh.T)
    ru_tok = ru.reshape(_NW, _DET, _TPW).transpose(0, 2, 1).reshape(_CTX, _DET)
    logits = unemb_tc(ru_tok, utab_bf)
    return logits.reshape(1, _CTX, _VOCAB)
